# Initial kernel scaffold; baseline (speedup 1.0000x reference)
import jax
import jax.numpy as jnp
from jax.experimental import pallas as pl


def kernel(x_in, speakers, W_ih, W_hh, b_ih, b_hh):
    raise NotImplementedError("write your pallas kernel here")


# speedup vs baseline: 4.8375x; 4.8375x over previous
import jax
import jax.numpy as jnp
from jax.experimental import pallas as pl
from jax.experimental.pallas import tpu as pltpu

S_MAX = 10


def _speaker_gru_kernel(x_ref, sp_ref, wih_ref, whh_ref, bih_ref, bhh_ref,
                        out_ref):
    blk, T, _ = x_ref.shape
    d = whh_ref.shape[0]

    wih = wih_ref[...]
    whh = whh_ref[...]
    bih = bih_ref[...]
    bhh = bhh_ref[...]

    sp = sp_ref[...]

    mem = [jnp.zeros((blk, d), jnp.float32) for _ in range(S_MAX)]
    for t in range(T):
        xt = x_ref[:, t, :]
        sidx = sp[:, t:t + 1]
        masks = [sidx == s for s in range(S_MAX)]
        h = jnp.where(masks[0], mem[0], 0.0)
        for s in range(1, S_MAX):
            h = jnp.where(masks[s], mem[s], h)
        gi = jnp.dot(xt, wih, preferred_element_type=jnp.float32) + bih
        gh = jnp.dot(h, whh, preferred_element_type=jnp.float32) + bhh
        r = jax.nn.sigmoid(gi[:, :d] + gh[:, :d])
        z = jax.nn.sigmoid(gi[:, d:2 * d] + gh[:, d:2 * d])
        n = jnp.tanh(gi[:, 2 * d:] + r * gh[:, 2 * d:])
        h_new = (1.0 - z) * n + z * h
        out_ref[:, t, :] = h_new
        mem = [jnp.where(masks[s], h_new, mem[s]) for s in range(S_MAX)]


def kernel(x_in, speakers, W_ih, W_hh, b_ih, b_hh):
    B, T, d_in = x_in.shape
    d = W_hh.shape[1]
    b_blk = 256

    sp = jnp.clip(speakers, 0, S_MAX - 1).astype(jnp.int32)
    wih = W_ih.T
    whh = W_hh.T
    bih = b_ih.reshape(1, -1)
    bhh = b_hh.reshape(1, -1)

    grid = (B // b_blk,)
    out = pl.pallas_call(
        _speaker_gru_kernel,
        grid=grid,
        in_specs=[
            pl.BlockSpec((b_blk, T, d_in), lambda i: (i, 0, 0)),
            pl.BlockSpec((b_blk, T), lambda i: (i, 0)),
            pl.BlockSpec((d_in, 3 * d), lambda i: (0, 0)),
            pl.BlockSpec((d, 3 * d), lambda i: (0, 0)),
            pl.BlockSpec((1, 3 * d), lambda i: (0, 0)),
            pl.BlockSpec((1, 3 * d), lambda i: (0, 0)),
        ],
        out_specs=pl.BlockSpec((b_blk, T, d), lambda i: (i, 0, 0)),
        out_shape=jax.ShapeDtypeStruct((B, T, d), x_in.dtype),
        compiler_params=pltpu.CompilerParams(
            dimension_semantics=("arbitrary",),
        ),
    )(x_in, sp, wih, whh, bih, bhh)
    return out
